# Initial kernel scaffold; baseline (speedup 1.0000x reference)
import jax
import jax.numpy as jnp
from jax.experimental import pallas as pl


def kernel(node_embs, edge_embs, edge_list, W1, W2, a1, a2):
    raise NotImplementedError("write your pallas kernel here")


# speedup vs baseline: 3.8065x; 3.8065x over previous
import functools

import jax
import jax.numpy as jnp
from jax import lax
from jax.experimental import pallas as pl
from jax.experimental.pallas import tpu as pltpu
from jax.experimental.pallas import tpu_sc as plsc

_N = 100000
_M = 100000
_AR = 4
_F = 128
_ALPHA = 0.2

_NC = 2
_NS = 16
_NW = _NC * _NS
_CHUNK = 128
_NCH = 25
_EPW = _NCH * _CHUNK
_MPAD = _NW * _EPW
_CSLICE = _AR * _N // _NS
_BUF = _AR * _EPW

_HI = lax.Precision.HIGHEST


def _u_body(x_ref, w1_ref, a1_ref, u_ref):
    wh = jnp.dot(x_ref[...], w1_ref[...], preferred_element_type=jnp.float32,
                 precision=_HI)
    lr = jnp.where(wh >= 0.0, wh, _ALPHA * wh)
    u_ref[...] = jnp.dot(lr, a1_ref[...], preferred_element_type=jnp.float32,
                         precision=_HI)


def _u_call(node_embs, W1, a1):
    B = 2000
    return pl.pallas_call(
        _u_body,
        grid=(_N // B,),
        in_specs=[
            pl.BlockSpec((B, _F), lambda i: (i, 0)),
            pl.BlockSpec((_F, _F), lambda i: (0, 0)),
            pl.BlockSpec((_F, 1), lambda i: (0, 0)),
        ],
        out_specs=pl.BlockSpec((B, 1), lambda i: (i, 0)),
        out_shape=jax.ShapeDtypeStruct((_N, 1), jnp.float32),
    )(node_embs, W1, a1)


def _sc_body(u_hbm, el_hbm, si_hbm, zeros_hbm, c2_hbm,
             ei_v, si_v, w_v, gu_v, c_sh, sem):
    cid = lax.axis_index("c")
    sid = lax.axis_index("s")
    wid = cid * _NS + sid

    pltpu.sync_copy(zeros_hbm, w_v)
    base_c = sid * _CSLICE
    pltpu.sync_copy(w_v, c_sh.at[pl.ds(base_c, _BUF)])
    pltpu.sync_copy(w_v.at[pl.ds(0, _CSLICE - _BUF)],
                    c_sh.at[pl.ds(base_c + _BUF, _CSLICE - _BUF)])

    for k in range(_AR):
        pltpu.sync_copy(el_hbm.at[k, wid], ei_v.at[pl.ds(k * _EPW, _EPW)])
        pltpu.sync_copy(si_hbm.at[k, wid], si_v.at[pl.ds(k * _NCH, _NCH)])

    def g_body(r, carry):
        sl = pl.ds(r * _CHUNK, _CHUNK)
        pltpu.sync_copy(u_hbm.at[ei_v.at[sl]], gu_v.at[sl])
        return carry

    lax.fori_loop(0, _AR * _NCH, g_body, 0)

    base = wid * _EPW
    iota16 = lax.iota(jnp.int32, 16)

    def chunk_body(c, carry):
        pos = c * 16 + iota16
        g = [plsc.load_gather(gu_v, [pos + k * _EPW]) for k in range(_AR)]
        mx = jnp.maximum(jnp.maximum(g[0], g[1]), jnp.maximum(g[2], g[3]))
        e = [jnp.exp(gk - mx) for gk in g]
        inv = 1.0 / (e[0] + e[1] + e[2] + e[3])
        valid = (base + c * 16 + iota16) < _M
        for k in range(_AR):
            plsc.store_scatter(w_v, [pos + k * _EPW],
                               jnp.where(valid, e[k] * inv, 0.0))
        return carry

    plsc.subcore_barrier()
    lax.fori_loop(0, _EPW // 16, chunk_body, 0)

    def scat_body(r, carry):
        pltpu.sync_copy(w_v.at[pl.ds(r * _CHUNK, _CHUNK)],
                        c_sh.at[si_v.at[r]], add=True)
        return carry

    lax.fori_loop(0, _AR * _NCH, scat_body, 0)

    plsc.subcore_barrier()

    hbase = cid * _AR * _N + sid * _CSLICE
    pltpu.sync_copy(c_sh.at[pl.ds(base_c, _BUF)], w_v)
    pltpu.sync_copy(w_v, c2_hbm.at[pl.ds(hbase, _BUF)])
    rem = _CSLICE - _BUF
    pltpu.sync_copy(c_sh.at[pl.ds(base_c + _BUF, rem)], w_v.at[pl.ds(0, rem)])
    pltpu.sync_copy(w_v.at[pl.ds(0, rem)], c2_hbm.at[pl.ds(hbase + _BUF, rem)])


def _sc_call(u_flat, el3, si4, zeros_c):
    mesh = plsc.VectorSubcoreMesh(core_axis_name="c", subcore_axis_name="s",
                                  num_cores=_NC, num_subcores=_NS)
    f = functools.partial(
        pl.kernel,
        out_type=jax.ShapeDtypeStruct((_NC * _AR * _N,), jnp.float32),
        mesh=mesh,
        compiler_params=pltpu.CompilerParams(needs_layout_passes=False),
        scratch_types=[
            pltpu.VMEM((_AR * _EPW,), jnp.int32),
            pltpu.VMEM((_AR * _NCH, _CHUNK), jnp.int32),
            pltpu.VMEM((_AR * _EPW,), jnp.float32),
            pltpu.VMEM((_AR * _EPW,), jnp.float32),
            pltpu.VMEM_SHARED((_AR * _N,), jnp.float32),
            pltpu.SemaphoreType.DMA,
        ],
    )(_sc_body)
    return f(u_flat, el3, si4, zeros_c)


def _out_body(ct_ref, x_ref, w1_ref, o_ref, acc_ref):
    i = pl.program_id(0)

    @pl.when(i == 0)
    def _():
        acc_ref[...] = jnp.zeros_like(acc_ref)

    ct = ct_ref[...]
    cs = ct[:, :_AR] + ct[:, _AR:]
    acc_ref[...] += lax.dot_general(
        cs, x_ref[...], (((0,), (0,)), ((), ())),
        preferred_element_type=jnp.float32, precision=_HI)

    @pl.when(i == pl.num_programs(0) - 1)
    def _():
        o_ref[...] = jnp.dot(acc_ref[...], w1_ref[...],
                             preferred_element_type=jnp.float32, precision=_HI)


def _out_call(ct, node_embs, W1):
    B = 2000
    return pl.pallas_call(
        _out_body,
        grid=(_N // B,),
        in_specs=[
            pl.BlockSpec((B, _NC * _AR), lambda i: (i, 0)),
            pl.BlockSpec((B, _F), lambda i: (i, 0)),
            pl.BlockSpec((_F, _F), lambda i: (0, 0)),
        ],
        out_specs=pl.BlockSpec((_AR, _F), lambda i: (0, 0)),
        out_shape=jax.ShapeDtypeStruct((_AR, _F), jnp.float32),
        scratch_shapes=[pltpu.VMEM((_AR, _F), jnp.float32)],
    )(ct, node_embs, W1)


def kernel(node_embs, edge_embs, edge_list, W1, W2, a1, a2):
    del edge_embs, W2, a2
    u = _u_call(node_embs, W1, a1)
    el = edge_list.astype(jnp.int32).T
    el = jnp.pad(el, ((0, 0), (0, _MPAD - _M)))
    el3 = el.reshape(_AR, _NW, _EPW)
    si = el + (jnp.arange(_AR, dtype=jnp.int32) * _N)[:, None]
    si4 = si.reshape(_AR, _NW, _NCH, _CHUNK)
    zeros_c = jnp.zeros((_BUF,), jnp.float32)
    c2 = _sc_call(u.reshape(_N), el3, si4, zeros_c)
    ct = c2.reshape(_NC * _AR, _N).T
    return _out_call(ct, node_embs, W1)


# speedup vs baseline: 4.2796x; 1.1243x over previous
import functools

import jax
import jax.numpy as jnp
from jax import lax
from jax.experimental import pallas as pl
from jax.experimental.pallas import tpu as pltpu
from jax.experimental.pallas import tpu_sc as plsc

_N = 100000
_M = 100000
_AR = 4
_F = 128
_ALPHA = 0.2

_NC = 2
_NS = 16
_NW = _NC * _NS
_CHUNK = 128
_NCH = 25
_EPW = _NCH * _CHUNK
_MPAD = _NW * _EPW
_CSLICE = _AR * _N // _NS
_BUF = _AR * _EPW

_HI = lax.Precision.HIGHEST
_HP = lax.Precision.HIGH
_UB = 2048
_UG = (_N + _UB - 1) // _UB


def _u_body(x_ref, w1_ref, a1r_ref, u_ref):
    wh = jnp.dot(x_ref[...], w1_ref[...], preferred_element_type=jnp.float32,
                 precision=_HI)
    lr = jnp.where(wh >= 0.0, wh, _ALPHA * wh)
    u_ref[...] = jnp.sum(lr * a1r_ref[...], axis=1)


def _u_call(node_embs, W1, a1r):
    return pl.pallas_call(
        _u_body,
        grid=(_UG,),
        in_specs=[
            pl.BlockSpec((_UB, _F), lambda i: (i, 0)),
            pl.BlockSpec((_F, _F), lambda i: (0, 0)),
            pl.BlockSpec((1, _F), lambda i: (0, 0)),
        ],
        out_specs=pl.BlockSpec((_UB,), lambda i: (i,)),
        out_shape=jax.ShapeDtypeStruct((_N,), jnp.float32),
    )(node_embs, W1, a1r)


def _sc_body(u_hbm, el_hbm, si_hbm, zeros_hbm, c2_hbm,
             ei_v, si_v, w_v, gu_v, c_sh, sem):
    cid = lax.axis_index("c")
    sid = lax.axis_index("s")
    wid = cid * _NS + sid

    pltpu.sync_copy(zeros_hbm, w_v)
    base_c = sid * _CSLICE
    pltpu.sync_copy(w_v, c_sh.at[pl.ds(base_c, _BUF)])
    pltpu.sync_copy(w_v.at[pl.ds(0, _CSLICE - _BUF)],
                    c_sh.at[pl.ds(base_c + _BUF, _CSLICE - _BUF)])

    for k in range(_AR):
        pltpu.sync_copy(el_hbm.at[k, wid], ei_v.at[pl.ds(k * _EPW, _EPW)])
        pltpu.sync_copy(si_hbm.at[k, wid], si_v.at[pl.ds(k * _NCH, _NCH)])

    def g_body(r, carry):
        sl = pl.ds(r * _CHUNK, _CHUNK)
        pltpu.sync_copy(u_hbm.at[ei_v.at[sl]], gu_v.at[sl])
        return carry

    lax.fori_loop(0, _AR * _NCH, g_body, 0)

    base = wid * _EPW
    iota16 = lax.iota(jnp.int32, 16)

    def chunk_body(c, carry):
        pos = c * 16 + iota16
        g = [plsc.load_gather(gu_v, [pos + k * _EPW]) for k in range(_AR)]
        mx = jnp.maximum(jnp.maximum(g[0], g[1]), jnp.maximum(g[2], g[3]))
        e = [jnp.exp(gk - mx) for gk in g]
        inv = 1.0 / (e[0] + e[1] + e[2] + e[3])
        valid = (base + c * 16 + iota16) < _M
        for k in range(_AR):
            plsc.store_scatter(w_v, [pos + k * _EPW],
                               jnp.where(valid, e[k] * inv, 0.0))
        return carry

    plsc.subcore_barrier()
    lax.fori_loop(0, _EPW // 16, chunk_body, 0)

    def scat_body(r, carry):
        pltpu.sync_copy(w_v.at[pl.ds(r * _CHUNK, _CHUNK)],
                        c_sh.at[si_v.at[r]], add=True)
        return carry

    lax.fori_loop(0, _AR * _NCH, scat_body, 0)

    plsc.subcore_barrier()

    hbase = cid * _AR * _N + sid * _CSLICE
    pltpu.sync_copy(c_sh.at[pl.ds(base_c, _BUF)], w_v)
    pltpu.sync_copy(w_v, c2_hbm.at[pl.ds(hbase, _BUF)])
    rem = _CSLICE - _BUF
    pltpu.sync_copy(c_sh.at[pl.ds(base_c + _BUF, rem)], w_v.at[pl.ds(0, rem)])
    pltpu.sync_copy(w_v.at[pl.ds(0, rem)], c2_hbm.at[pl.ds(hbase + _BUF, rem)])


def _sc_call(u_flat, el3, si4, zeros_c):
    mesh = plsc.VectorSubcoreMesh(core_axis_name="c", subcore_axis_name="s",
                                  num_cores=_NC, num_subcores=_NS)
    f = functools.partial(
        pl.kernel,
        out_type=jax.ShapeDtypeStruct((_NC * _AR * _N,), jnp.float32),
        mesh=mesh,
        compiler_params=pltpu.CompilerParams(needs_layout_passes=False),
        scratch_types=[
            pltpu.VMEM((_AR * _EPW,), jnp.int32),
            pltpu.VMEM((_AR * _NCH, _CHUNK), jnp.int32),
            pltpu.VMEM((_AR * _EPW,), jnp.float32),
            pltpu.VMEM((_AR * _EPW,), jnp.float32),
            pltpu.VMEM_SHARED((_AR * _N,), jnp.float32),
            pltpu.SemaphoreType.DMA,
        ],
    )(_sc_body)
    return f(u_flat, el3, si4, zeros_c)


def _out_body(c_ref, x_ref, w1_ref, o_ref, acc_ref):
    i = pl.program_id(0)

    @pl.when(i == 0)
    def _():
        acc_ref[...] = jnp.zeros_like(acc_ref)

    valid = _N - i * _UB
    c = c_ref[...]
    x = x_ref[...]
    lane = lax.broadcasted_iota(jnp.int32, (1, _UB), 1)
    c = jnp.where(lane < valid, c, 0.0)
    x = jnp.where(lane.reshape(_UB, 1) < valid, x, 0.0)
    acc_ref[...] += lax.dot_general(
        c, x, (((1,), (0,)), ((), ())),
        preferred_element_type=jnp.float32, precision=_HI)

    @pl.when(i == pl.num_programs(0) - 1)
    def _():
        c8 = acc_ref[...]
        cs = c8[:_AR] + c8[_AR:]
        o_ref[...] = jnp.dot(cs, w1_ref[...], preferred_element_type=jnp.float32,
                             precision=_HI)


def _out_call(c2, node_embs, W1):
    return pl.pallas_call(
        _out_body,
        grid=(_UG,),
        in_specs=[
            pl.BlockSpec((_NC * _AR, _UB), lambda i: (0, i)),
            pl.BlockSpec((_UB, _F), lambda i: (i, 0)),
            pl.BlockSpec((_F, _F), lambda i: (0, 0)),
        ],
        out_specs=pl.BlockSpec((_AR, _F), lambda i: (0, 0)),
        out_shape=jax.ShapeDtypeStruct((_AR, _F), jnp.float32),
        scratch_shapes=[pltpu.VMEM((_NC * _AR, _F), jnp.float32)],
    )(c2, node_embs, W1)


def kernel(node_embs, edge_embs, edge_list, W1, W2, a1, a2):
    del edge_embs, W2, a2
    u = _u_call(node_embs, W1, a1.reshape(1, _F))
    el = edge_list.astype(jnp.int32).T
    el = jnp.pad(el, ((0, 0), (0, _MPAD - _M)))
    el3 = el.reshape(_AR, _NW, _EPW)
    si = el + (jnp.arange(_AR, dtype=jnp.int32) * _N)[:, None]
    si4 = si.reshape(_AR, _NW, _NCH, _CHUNK)
    zeros_c = jnp.zeros((_BUF,), jnp.float32)
    c2 = _sc_call(u, el3, si4, zeros_c)
    return _out_call(c2.reshape(_NC * _AR, _N), node_embs, W1)


# speedup vs baseline: 4.4564x; 1.0413x over previous
import functools

import jax
import jax.numpy as jnp
from jax import lax
from jax.experimental import pallas as pl
from jax.experimental.pallas import tpu as pltpu
from jax.experimental.pallas import tpu_sc as plsc

_N = 100000
_M = 100000
_AR = 4
_F = 128
_ALPHA = 0.2

_NC = 2
_NS = 16
_NW = _NC * _NS
_CHUNK = 128
_NCH = 25
_EPW = _NCH * _CHUNK
_MPAD = _NW * _EPW
_CSLICE = _AR * _N // _NS
_BUF = _AR * _EPW

_HI = lax.Precision.HIGHEST
_HP = lax.Precision.HIGH
_UB = 2048
_UG = (_N + _UB - 1) // _UB


def _u_body(x_ref, w1_ref, a1b_ref, u_ref):
    wh = jnp.dot(x_ref[...], w1_ref[...], preferred_element_type=jnp.float32,
                 precision=_HI)
    lr = jnp.where(wh >= 0.0, wh, _ALPHA * wh)
    u_ref[...] = jnp.dot(lr, a1b_ref[...], preferred_element_type=jnp.float32,
                         precision=_HI)


def _u_call(node_embs, W1, a1b):
    return pl.pallas_call(
        _u_body,
        grid=(_UG,),
        in_specs=[
            pl.BlockSpec((_UB, _F), lambda i: (i, 0)),
            pl.BlockSpec((_F, _F), lambda i: (0, 0)),
            pl.BlockSpec((_F, _F), lambda i: (0, 0)),
        ],
        out_specs=pl.BlockSpec((_UB, _F), lambda i: (i, 0)),
        out_shape=jax.ShapeDtypeStruct((_UG * _UB, _F), jnp.float32),
    )(node_embs, W1, a1b)


def _sc_body(u_hbm, el_hbm, si_hbm, zeros_hbm, c2_hbm,
             ei_v, si_v, w_v, gu_v, c_sh, sem):
    cid = lax.axis_index("c")
    sid = lax.axis_index("s")
    wid = cid * _NS + sid

    pltpu.sync_copy(zeros_hbm, w_v)
    base_c = sid * _CSLICE
    pltpu.sync_copy(w_v, c_sh.at[pl.ds(base_c, _BUF)])
    pltpu.sync_copy(w_v.at[pl.ds(0, _CSLICE - _BUF)],
                    c_sh.at[pl.ds(base_c + _BUF, _CSLICE - _BUF)])

    for k in range(_AR):
        pltpu.sync_copy(el_hbm.at[k, wid], ei_v.at[pl.ds(k * _EPW, _EPW)])
        pltpu.sync_copy(si_hbm.at[k, wid], si_v.at[pl.ds(k * _NCH, _NCH)])

    def g_body(r, carry):
        sl = pl.ds(r * _CHUNK, _CHUNK)
        pltpu.sync_copy(u_hbm.at[ei_v.at[sl]], gu_v.at[sl])
        return carry

    lax.fori_loop(0, _AR * _NCH, g_body, 0)

    base = wid * _EPW
    iota16 = lax.iota(jnp.int32, 16)

    def chunk_body(c, carry):
        pos = c * 16 + iota16
        g = [plsc.load_gather(gu_v, [pos + k * _EPW]) for k in range(_AR)]
        mx = jnp.maximum(jnp.maximum(g[0], g[1]), jnp.maximum(g[2], g[3]))
        e = [jnp.exp(gk - mx) for gk in g]
        inv = 1.0 / (e[0] + e[1] + e[2] + e[3])
        valid = (base + c * 16 + iota16) < _M
        for k in range(_AR):
            plsc.store_scatter(w_v, [pos + k * _EPW],
                               jnp.where(valid, e[k] * inv, 0.0))
        return carry

    plsc.subcore_barrier()
    lax.fori_loop(0, _EPW // 16, chunk_body, 0)

    def scat_body(r, carry):
        pltpu.sync_copy(w_v.at[pl.ds(r * _CHUNK, _CHUNK)],
                        c_sh.at[si_v.at[r]], add=True)
        return carry

    lax.fori_loop(0, _AR * _NCH, scat_body, 0)

    plsc.subcore_barrier()

    hbase = cid * _AR * _N + sid * _CSLICE
    pltpu.sync_copy(c_sh.at[pl.ds(base_c, _BUF)], w_v)
    pltpu.sync_copy(w_v, c2_hbm.at[pl.ds(hbase, _BUF)])
    rem = _CSLICE - _BUF
    pltpu.sync_copy(c_sh.at[pl.ds(base_c + _BUF, rem)], w_v.at[pl.ds(0, rem)])
    pltpu.sync_copy(w_v.at[pl.ds(0, rem)], c2_hbm.at[pl.ds(hbase + _BUF, rem)])


def _sc_call(u_flat, el3, si4, zeros_c):
    mesh = plsc.VectorSubcoreMesh(core_axis_name="c", subcore_axis_name="s",
                                  num_cores=_NC, num_subcores=_NS)
    f = functools.partial(
        pl.kernel,
        out_type=jax.ShapeDtypeStruct((_NC * _AR * _N,), jnp.float32),
        mesh=mesh,
        compiler_params=pltpu.CompilerParams(needs_layout_passes=False),
        scratch_types=[
            pltpu.VMEM((_AR * _EPW,), jnp.int32),
            pltpu.VMEM((_AR * _NCH, _CHUNK), jnp.int32),
            pltpu.VMEM((_AR * _EPW,), jnp.float32),
            pltpu.VMEM((_AR * _EPW,), jnp.float32),
            pltpu.VMEM_SHARED((_AR * _N,), jnp.float32),
            pltpu.SemaphoreType.DMA,
        ],
    )(_sc_body)
    return f(u_flat, el3, si4, zeros_c)


def _out_body(c_ref, x_ref, w1_ref, o_ref, acc_ref):
    i = pl.program_id(0)

    @pl.when(i == 0)
    def _():
        acc_ref[...] = jnp.zeros_like(acc_ref)

    valid = _N - i * _UB
    c = c_ref[...]
    x = x_ref[...]
    lane = lax.broadcasted_iota(jnp.int32, (1, _UB), 1)
    c = jnp.where(lane < valid, c, 0.0)
    x = jnp.where(lane.reshape(_UB, 1) < valid, x, 0.0)
    acc_ref[...] += lax.dot_general(
        c, x, (((1,), (0,)), ((), ())),
        preferred_element_type=jnp.float32, precision=_HI)

    @pl.when(i == pl.num_programs(0) - 1)
    def _():
        c8 = acc_ref[...]
        cs = c8[:_AR] + c8[_AR:]
        o_ref[...] = jnp.dot(cs, w1_ref[...], preferred_element_type=jnp.float32,
                             precision=_HI)


def _out_call(c2, node_embs, W1):
    return pl.pallas_call(
        _out_body,
        grid=(_UG,),
        in_specs=[
            pl.BlockSpec((_NC * _AR, _UB), lambda i: (0, i)),
            pl.BlockSpec((_UB, _F), lambda i: (i, 0)),
            pl.BlockSpec((_F, _F), lambda i: (0, 0)),
        ],
        out_specs=pl.BlockSpec((_AR, _F), lambda i: (0, 0)),
        out_shape=jax.ShapeDtypeStruct((_AR, _F), jnp.float32),
        scratch_shapes=[pltpu.VMEM((_NC * _AR, _F), jnp.float32)],
    )(c2, node_embs, W1)


def kernel(node_embs, edge_embs, edge_list, W1, W2, a1, a2):
    del edge_embs, W2, a2
    a1b = a1.reshape(_F, 1) * jnp.ones((1, _F), jnp.float32)
    u = _u_call(node_embs, W1, a1b)
    el = edge_list.astype(jnp.int32).T
    el = jnp.pad(el, ((0, 0), (0, _MPAD - _M)))
    el3 = (el * _F).reshape(_AR, _NW, _EPW)
    si = el + (jnp.arange(_AR, dtype=jnp.int32) * _N)[:, None]
    si4 = si.reshape(_AR, _NW, _NCH, _CHUNK)
    zeros_c = jnp.zeros((_BUF,), jnp.float32)
    c2 = _sc_call(u.reshape(-1), el3, si4, zeros_c)
    return _out_call(c2.reshape(_NC * _AR, _N), node_embs, W1)


# speedup vs baseline: 5.2993x; 1.1892x over previous
import functools

import jax
import jax.numpy as jnp
from jax import lax
from jax.experimental import pallas as pl
from jax.experimental.pallas import tpu as pltpu
from jax.experimental.pallas import tpu_sc as plsc

_N = 100000
_M = 100000
_AR = 4
_F = 128
_ALPHA = 0.2

_NC = 2
_NS = 16
_NW = _NC * _NS
_CHUNK = 128
_NCH = 25
_EPW = _NCH * _CHUNK
_MPAD = _NW * _EPW
_CSLICE = _AR * _N // _NS
_BUF = _AR * _EPW

_HI = lax.Precision.HIGHEST
_HP = lax.Precision.HIGH
_UB = 2048
_UG = (_N + _UB - 1) // _UB
_NP = _UG * _UB


def _u_body(x_ref, w1_ref, a1r_ref, u_ref):
    wh = jnp.dot(x_ref[...], w1_ref[...], preferred_element_type=jnp.float32,
                 precision=_HI)
    lr = jnp.where(wh >= 0.0, wh, _ALPHA * wh)
    u_ref[...] = jnp.sum(lr * a1r_ref[...], axis=1)


def _u_call(node_embs, W1, a1r):
    return pl.pallas_call(
        _u_body,
        grid=(_UG,),
        in_specs=[
            pl.BlockSpec((_UB, _F), lambda i: (i, 0)),
            pl.BlockSpec((_F, _F), lambda i: (0, 0)),
            pl.BlockSpec((1, _F), lambda i: (0, 0)),
        ],
        out_specs=pl.BlockSpec((_UB,), lambda i: (i,)),
        out_shape=jax.ShapeDtypeStruct((_NP,), jnp.float32),
    )(node_embs, W1, a1r)


def _sc_body(u_hbm, el_hbm, si_hbm, zeros_hbm, c2_hbm,
             ei_v, si_v, w_v, gu_v, c_sh, u_sh, sem):
    cid = lax.axis_index("c")
    sid = lax.axis_index("s")
    wid = cid * _NS + sid

    pltpu.sync_copy(zeros_hbm, w_v)
    base_c = sid * _CSLICE
    pltpu.sync_copy(w_v, c_sh.at[pl.ds(base_c, _BUF)])
    pltpu.sync_copy(w_v.at[pl.ds(0, _CSLICE - _BUF)],
                    c_sh.at[pl.ds(base_c + _BUF, _CSLICE - _BUF)])

    upt = _NP // _NS
    pltpu.sync_copy(u_hbm.at[pl.ds(sid * upt, upt)], gu_v.at[pl.ds(0, upt)])
    pltpu.sync_copy(gu_v.at[pl.ds(0, upt)], u_sh.at[pl.ds(sid * upt, upt)])

    for k in range(_AR):
        pltpu.sync_copy(el_hbm.at[k, wid], ei_v.at[pl.ds(k * _EPW, _EPW)])
        pltpu.sync_copy(si_hbm.at[k, wid], si_v.at[pl.ds(k * _NCH, _NCH)])

    plsc.subcore_barrier()

    def g_body(r, carry):
        sl = pl.ds(r * _CHUNK, _CHUNK)
        pltpu.sync_copy(u_sh.at[ei_v.at[sl]], gu_v.at[sl])
        return carry

    lax.fori_loop(0, _AR * _NCH, g_body, 0)

    base = wid * _EPW
    iota16 = lax.iota(jnp.int32, 16)

    def chunk_body(c, carry):
        pos = c * 16 + iota16
        g = [plsc.load_gather(gu_v, [pos + k * _EPW]) for k in range(_AR)]
        mx = jnp.maximum(jnp.maximum(g[0], g[1]), jnp.maximum(g[2], g[3]))
        e = [jnp.exp(gk - mx) for gk in g]
        inv = 1.0 / (e[0] + e[1] + e[2] + e[3])
        valid = (base + c * 16 + iota16) < _M
        for k in range(_AR):
            plsc.store_scatter(w_v, [pos + k * _EPW],
                               jnp.where(valid, e[k] * inv, 0.0))
        return carry

    lax.fori_loop(0, _EPW // 16, chunk_body, 0)

    def scat_body(r, carry):
        pltpu.sync_copy(w_v.at[pl.ds(r * _CHUNK, _CHUNK)],
                        c_sh.at[si_v.at[r]], add=True)
        return carry

    lax.fori_loop(0, _AR * _NCH, scat_body, 0)

    plsc.subcore_barrier()

    hbase = cid * _AR * _N + sid * _CSLICE
    pltpu.sync_copy(c_sh.at[pl.ds(base_c, _BUF)], w_v)
    pltpu.sync_copy(w_v, c2_hbm.at[pl.ds(hbase, _BUF)])
    rem = _CSLICE - _BUF
    pltpu.sync_copy(c_sh.at[pl.ds(base_c + _BUF, rem)], w_v.at[pl.ds(0, rem)])
    pltpu.sync_copy(w_v.at[pl.ds(0, rem)], c2_hbm.at[pl.ds(hbase + _BUF, rem)])


def _sc_call(u_flat, el3, si4, zeros_c):
    mesh = plsc.VectorSubcoreMesh(core_axis_name="c", subcore_axis_name="s",
                                  num_cores=_NC, num_subcores=_NS)
    f = functools.partial(
        pl.kernel,
        out_type=jax.ShapeDtypeStruct((_NC * _AR * _N,), jnp.float32),
        mesh=mesh,
        compiler_params=pltpu.CompilerParams(needs_layout_passes=False),
        scratch_types=[
            pltpu.VMEM((_AR * _EPW,), jnp.int32),
            pltpu.VMEM((_AR * _NCH, _CHUNK), jnp.int32),
            pltpu.VMEM((_AR * _EPW,), jnp.float32),
            pltpu.VMEM((_AR * _EPW,), jnp.float32),
            pltpu.VMEM_SHARED((_AR * _N,), jnp.float32),
            pltpu.VMEM_SHARED((_NP,), jnp.float32),
            pltpu.SemaphoreType.DMA,
        ],
    )(_sc_body)
    return f(u_flat, el3, si4, zeros_c)


def _out_body(c_ref, x_ref, w1_ref, o_ref, acc_ref):
    i = pl.program_id(0)

    @pl.when(i == 0)
    def _():
        acc_ref[...] = jnp.zeros_like(acc_ref)

    valid = _N - i * _UB
    c = c_ref[...]
    x = x_ref[...]
    lane = lax.broadcasted_iota(jnp.int32, (1, _UB), 1)
    c = jnp.where(lane < valid, c, 0.0)
    x = jnp.where(lane.reshape(_UB, 1) < valid, x, 0.0)
    acc_ref[...] += lax.dot_general(
        c, x, (((1,), (0,)), ((), ())),
        preferred_element_type=jnp.float32, precision=_HI)

    @pl.when(i == pl.num_programs(0) - 1)
    def _():
        c8 = acc_ref[...]
        cs = c8[:_AR] + c8[_AR:]
        o_ref[...] = jnp.dot(cs, w1_ref[...], preferred_element_type=jnp.float32,
                             precision=_HI)


def _out_call(c2, node_embs, W1):
    return pl.pallas_call(
        _out_body,
        grid=(_UG,),
        in_specs=[
            pl.BlockSpec((_NC * _AR, _UB), lambda i: (0, i)),
            pl.BlockSpec((_UB, _F), lambda i: (i, 0)),
            pl.BlockSpec((_F, _F), lambda i: (0, 0)),
        ],
        out_specs=pl.BlockSpec((_AR, _F), lambda i: (0, 0)),
        out_shape=jax.ShapeDtypeStruct((_AR, _F), jnp.float32),
        scratch_shapes=[pltpu.VMEM((_NC * _AR, _F), jnp.float32)],
    )(c2, node_embs, W1)


def kernel(node_embs, edge_embs, edge_list, W1, W2, a1, a2):
    del edge_embs, W2, a2
    u = _u_call(node_embs, W1, a1.reshape(1, _F))
    el = edge_list.astype(jnp.int32).T
    el = jnp.pad(el, ((0, 0), (0, _MPAD - _M)))
    el3 = el.reshape(_AR, _NW, _EPW)
    si = el + (jnp.arange(_AR, dtype=jnp.int32) * _N)[:, None]
    si4 = si.reshape(_AR, _NW, _NCH, _CHUNK)
    zeros_c = jnp.zeros((_BUF,), jnp.float32)
    c2 = _sc_call(u, el3, si4, zeros_c)
    return _out_call(c2.reshape(_NC * _AR, _N), node_embs, W1)


# speedup vs baseline: 5.5670x; 1.0505x over previous
import functools

import jax
import jax.numpy as jnp
from jax import lax
from jax.experimental import pallas as pl
from jax.experimental.pallas import tpu as pltpu
from jax.experimental.pallas import tpu_sc as plsc

_N = 100000
_M = 100000
_AR = 4
_F = 128
_ALPHA = 0.2

_NC = 2
_NS = 16
_NW = _NC * _NS
_CHUNK = 128
_NCH = 25
_EPW = _NCH * _CHUNK
_MPAD = _NW * _EPW
_CSLICE = _AR * _N // _NS
_BUF = _AR * _EPW

_HI = lax.Precision.HIGHEST
_HP = lax.Precision.HIGH
_UB = 2048
_UG = (_N + _UB - 1) // _UB
_NP = _UG * _UB


def _u_body(x_ref, w1_ref, a1r_ref, u_ref):
    wh = jnp.dot(x_ref[...], w1_ref[...], preferred_element_type=jnp.float32,
                 precision=_HI)
    lr = jnp.where(wh >= 0.0, wh, _ALPHA * wh)
    u_ref[...] = lax.dot_general(a1r_ref[...], lr, (((1,), (1,)), ((), ())),
                                 preferred_element_type=jnp.float32,
                                 precision=_HI)


def _u_call(node_embs, W1, a1r):
    return pl.pallas_call(
        _u_body,
        grid=(_UG,),
        in_specs=[
            pl.BlockSpec((_UB, _F), lambda i: (i, 0)),
            pl.BlockSpec((_F, _F), lambda i: (0, 0)),
            pl.BlockSpec((8, _F), lambda i: (0, 0)),
        ],
        out_specs=pl.BlockSpec((8, _UB), lambda i: (i, 0)),
        out_shape=jax.ShapeDtypeStruct((_UG * 8, _UB), jnp.float32),
    )(node_embs, W1, a1r)


def _sc_body(u_hbm, el_hbm, si_hbm, zeros_hbm, c2_hbm,
             ei_v, si_v, w_v, gu_v, c_sh, u_sh, sem):
    cid = lax.axis_index("c")
    sid = lax.axis_index("s")
    wid = cid * _NS + sid

    pltpu.sync_copy(zeros_hbm, w_v)
    base_c = sid * _CSLICE
    pltpu.sync_copy(w_v, c_sh.at[pl.ds(base_c, _BUF)])
    pltpu.sync_copy(w_v.at[pl.ds(0, _CSLICE - _BUF)],
                    c_sh.at[pl.ds(base_c + _BUF, _CSLICE - _BUF)])

    upt = _NP // _NS
    pltpu.sync_copy(u_hbm.at[pl.ds(sid * upt, upt)], gu_v.at[pl.ds(0, upt)])
    pltpu.sync_copy(gu_v.at[pl.ds(0, upt)], u_sh.at[pl.ds(sid * upt, upt)])

    for k in range(_AR):
        pltpu.sync_copy(el_hbm.at[k, wid], ei_v.at[pl.ds(k * _EPW, _EPW)])
        pltpu.sync_copy(si_hbm.at[k, wid], si_v.at[pl.ds(k * _NCH, _NCH)])

    plsc.subcore_barrier()

    def g_body(r, carry):
        sl = pl.ds(r * _CHUNK, _CHUNK)
        pltpu.sync_copy(u_sh.at[ei_v.at[sl]], gu_v.at[sl])
        return carry

    lax.fori_loop(0, _AR * _NCH, g_body, 0)

    base = wid * _EPW
    iota16 = lax.iota(jnp.int32, 16)

    def chunk_body(c, carry):
        pos = c * 16 + iota16
        g = [plsc.load_gather(gu_v, [pos + k * _EPW]) for k in range(_AR)]
        mx = jnp.maximum(jnp.maximum(g[0], g[1]), jnp.maximum(g[2], g[3]))
        e = [jnp.exp(gk - mx) for gk in g]
        inv = 1.0 / (e[0] + e[1] + e[2] + e[3])
        valid = (base + c * 16 + iota16) < _M
        for k in range(_AR):
            plsc.store_scatter(w_v, [pos + k * _EPW],
                               jnp.where(valid, e[k] * inv, 0.0))
        return carry

    lax.fori_loop(0, _EPW // 16, chunk_body, 0)

    def scat_body(r, carry):
        pltpu.sync_copy(w_v.at[pl.ds(r * _CHUNK, _CHUNK)],
                        c_sh.at[si_v.at[r]], add=True)
        return carry

    lax.fori_loop(0, _AR * _NCH, scat_body, 0)

    plsc.subcore_barrier()

    hbase = cid * _AR * _N + sid * _CSLICE
    pltpu.sync_copy(c_sh.at[pl.ds(base_c, _BUF)], w_v)
    pltpu.sync_copy(w_v, c2_hbm.at[pl.ds(hbase, _BUF)])
    rem = _CSLICE - _BUF
    pltpu.sync_copy(c_sh.at[pl.ds(base_c + _BUF, rem)], w_v.at[pl.ds(0, rem)])
    pltpu.sync_copy(w_v.at[pl.ds(0, rem)], c2_hbm.at[pl.ds(hbase + _BUF, rem)])


def _sc_call(u_flat, el3, si4, zeros_c):
    mesh = plsc.VectorSubcoreMesh(core_axis_name="c", subcore_axis_name="s",
                                  num_cores=_NC, num_subcores=_NS)
    f = functools.partial(
        pl.kernel,
        out_type=jax.ShapeDtypeStruct((_NC * _AR * _N,), jnp.float32),
        mesh=mesh,
        compiler_params=pltpu.CompilerParams(needs_layout_passes=False),
        scratch_types=[
            pltpu.VMEM((_AR * _EPW,), jnp.int32),
            pltpu.VMEM((_AR * _NCH, _CHUNK), jnp.int32),
            pltpu.VMEM((_AR * _EPW,), jnp.float32),
            pltpu.VMEM((_AR * _EPW,), jnp.float32),
            pltpu.VMEM_SHARED((_AR * _N,), jnp.float32),
            pltpu.VMEM_SHARED((_NP,), jnp.float32),
            pltpu.SemaphoreType.DMA,
        ],
    )(_sc_body)
    return f(u_flat, el3, si4, zeros_c)


def _out_body(c_ref, x_ref, w1_ref, o_ref, acc_ref):
    i = pl.program_id(0)

    @pl.when(i == 0)
    def _():
        acc_ref[...] = jnp.zeros_like(acc_ref)

    valid = _N - i * _UB
    c = c_ref[...]
    x = x_ref[...]
    lane = lax.broadcasted_iota(jnp.int32, (1, _UB), 1)
    c = jnp.where(lane < valid, c, 0.0)
    x = jnp.where(lane.reshape(_UB, 1) < valid, x, 0.0)
    acc_ref[...] += lax.dot_general(
        c, x, (((1,), (0,)), ((), ())),
        preferred_element_type=jnp.float32, precision=_HI)

    @pl.when(i == pl.num_programs(0) - 1)
    def _():
        c8 = acc_ref[...]
        cs = c8[:_AR] + c8[_AR:]
        o_ref[...] = jnp.dot(cs, w1_ref[...], preferred_element_type=jnp.float32,
                             precision=_HI)


def _out_call(c2, node_embs, W1):
    return pl.pallas_call(
        _out_body,
        grid=(_UG,),
        in_specs=[
            pl.BlockSpec((_NC * _AR, _UB), lambda i: (0, i)),
            pl.BlockSpec((_UB, _F), lambda i: (i, 0)),
            pl.BlockSpec((_F, _F), lambda i: (0, 0)),
        ],
        out_specs=pl.BlockSpec((_AR, _F), lambda i: (0, 0)),
        out_shape=jax.ShapeDtypeStruct((_AR, _F), jnp.float32),
        scratch_shapes=[pltpu.VMEM((_NC * _AR, _F), jnp.float32)],
    )(c2, node_embs, W1)


def kernel(node_embs, edge_embs, edge_list, W1, W2, a1, a2):
    del edge_embs, W2, a2
    a1r = jnp.broadcast_to(a1.reshape(1, _F), (8, _F))
    u8 = _u_call(node_embs, W1, a1r)
    u = u8.reshape(_UG, 8, _UB)[:, 0, :].reshape(_NP)
    el = edge_list.astype(jnp.int32).T
    el = jnp.pad(el, ((0, 0), (0, _MPAD - _M)))
    el3 = el.reshape(_AR, _NW, _EPW)
    si = el + (jnp.arange(_AR, dtype=jnp.int32) * _N)[:, None]
    si4 = si.reshape(_AR, _NW, _NCH, _CHUNK)
    zeros_c = jnp.zeros((_BUF,), jnp.float32)
    c2 = _sc_call(u, el3, si4, zeros_c)
    return _out_call(c2.reshape(_NC * _AR, _N), node_embs, W1)


# speedup vs baseline: 7.6177x; 1.3684x over previous
import functools

import jax
import jax.numpy as jnp
from jax import lax
from jax.experimental import pallas as pl
from jax.experimental.pallas import tpu as pltpu
from jax.experimental.pallas import tpu_sc as plsc

_N = 100000
_M = 100000
_AR = 4
_F = 128
_ALPHA = 0.2

_NC = 2
_NS = 16
_NW = _NC * _NS
_CHUNK = 128
_NCH = 25
_EPW = _NCH * _CHUNK
_MPAD = _NW * _EPW
_CSLICE = _AR * _N // _NS
_BUF = _AR * _EPW

_HI = lax.Precision.HIGHEST
_HP = lax.Precision.HIGH
_UB = 2048
_UG = (_N + _UB - 1) // _UB
_NP = _UG * _UB


def _split(x):
    hi = x.astype(jnp.bfloat16)
    lo = (x - hi.astype(jnp.float32)).astype(jnp.bfloat16)
    return hi, lo


def _u_body(x_ref, w1_ref, a1r_ref, u_ref):
    xh, xl = _split(x_ref[...])
    wh_, wl_ = _split(w1_ref[...])
    d = functools.partial(jnp.dot, preferred_element_type=jnp.float32)
    wh = d(xh, wh_) + d(xh, wl_) + d(xl, wh_)
    lr = jnp.where(wh >= 0.0, wh, _ALPHA * wh)
    lh, ll = _split(lr)
    ah, al = _split(a1r_ref[...])
    dg = functools.partial(lax.dot_general,
                           dimension_numbers=(((1,), (1,)), ((), ())),
                           preferred_element_type=jnp.float32)
    u_ref[...] = dg(ah, lh) + dg(al, lh) + dg(ah, ll)


def _u_call(node_embs, W1, a1r):
    return pl.pallas_call(
        _u_body,
        grid=(_UG,),
        in_specs=[
            pl.BlockSpec((_UB, _F), lambda i: (i, 0)),
            pl.BlockSpec((_F, _F), lambda i: (0, 0)),
            pl.BlockSpec((8, _F), lambda i: (0, 0)),
        ],
        out_specs=pl.BlockSpec((8, _UB), lambda i: (i, 0)),
        out_shape=jax.ShapeDtypeStruct((_UG * 8, _UB), jnp.float32),
    )(node_embs, W1, a1r)


def _sc_body(u_hbm, el_hbm, si_hbm, zeros_hbm, c2_hbm,
             ei_v, si_v, w_v, gu_v, c_sh, u_sh, sem):
    cid = lax.axis_index("c")
    sid = lax.axis_index("s")
    wid = cid * _NS + sid

    pltpu.sync_copy(zeros_hbm, w_v)
    base_c = sid * _CSLICE
    pltpu.sync_copy(w_v, c_sh.at[pl.ds(base_c, _BUF)])
    pltpu.sync_copy(w_v.at[pl.ds(0, _CSLICE - _BUF)],
                    c_sh.at[pl.ds(base_c + _BUF, _CSLICE - _BUF)])

    upt = _NP // _NS
    pltpu.sync_copy(u_hbm.at[pl.ds(sid * upt, upt)], gu_v.at[pl.ds(0, upt)])
    pltpu.sync_copy(gu_v.at[pl.ds(0, upt)], u_sh.at[pl.ds(sid * upt, upt)])

    for k in range(_AR):
        pltpu.sync_copy(el_hbm.at[k, wid], ei_v.at[pl.ds(k * _EPW, _EPW)])
        pltpu.sync_copy(si_hbm.at[k, wid], si_v.at[pl.ds(k * _NCH, _NCH)])

    plsc.subcore_barrier()

    def g_body(r, carry):
        sl = pl.ds(r * _CHUNK, _CHUNK)
        pltpu.sync_copy(u_sh.at[ei_v.at[sl]], gu_v.at[sl])
        return carry

    lax.fori_loop(0, _AR * _NCH, g_body, 0)

    base = wid * _EPW
    iota16 = lax.iota(jnp.int32, 16)

    def chunk_body(c, carry):
        pos = c * 16 + iota16
        g = [plsc.load_gather(gu_v, [pos + k * _EPW]) for k in range(_AR)]
        mx = jnp.maximum(jnp.maximum(g[0], g[1]), jnp.maximum(g[2], g[3]))
        e = [jnp.exp(gk - mx) for gk in g]
        inv = 1.0 / (e[0] + e[1] + e[2] + e[3])
        valid = (base + c * 16 + iota16) < _M
        for k in range(_AR):
            plsc.store_scatter(w_v, [pos + k * _EPW],
                               jnp.where(valid, e[k] * inv, 0.0))
        return carry

    lax.fori_loop(0, _EPW // 16, chunk_body, 0)

    def scat_body(r, carry):
        pltpu.sync_copy(w_v.at[pl.ds(r * _CHUNK, _CHUNK)],
                        c_sh.at[si_v.at[r]], add=True)
        return carry

    lax.fori_loop(0, _AR * _NCH, scat_body, 0)

    plsc.subcore_barrier()

    hbase = cid * _AR * _N + sid * _CSLICE
    pltpu.sync_copy(c_sh.at[pl.ds(base_c, _BUF)], w_v)
    pltpu.sync_copy(w_v, c2_hbm.at[pl.ds(hbase, _BUF)])
    rem = _CSLICE - _BUF
    pltpu.sync_copy(c_sh.at[pl.ds(base_c + _BUF, rem)], w_v.at[pl.ds(0, rem)])
    pltpu.sync_copy(w_v.at[pl.ds(0, rem)], c2_hbm.at[pl.ds(hbase + _BUF, rem)])


def _sc_call(u_flat, el3, si4, zeros_c):
    mesh = plsc.VectorSubcoreMesh(core_axis_name="c", subcore_axis_name="s",
                                  num_cores=_NC, num_subcores=_NS)
    f = functools.partial(
        pl.kernel,
        out_type=jax.ShapeDtypeStruct((_NC * _AR * _N,), jnp.float32),
        mesh=mesh,
        compiler_params=pltpu.CompilerParams(needs_layout_passes=False),
        scratch_types=[
            pltpu.VMEM((_AR * _EPW,), jnp.int32),
            pltpu.VMEM((_AR * _NCH, _CHUNK), jnp.int32),
            pltpu.VMEM((_AR * _EPW,), jnp.float32),
            pltpu.VMEM((_AR * _EPW,), jnp.float32),
            pltpu.VMEM_SHARED((_AR * _N,), jnp.float32),
            pltpu.VMEM_SHARED((_NP,), jnp.float32),
            pltpu.SemaphoreType.DMA,
        ],
    )(_sc_body)
    return f(u_flat, el3, si4, zeros_c)


def _out_body(c_ref, x_ref, w1_ref, o_ref, acc_ref):
    i = pl.program_id(0)

    @pl.when(i == 0)
    def _():
        acc_ref[...] = jnp.zeros_like(acc_ref)

    valid = _N - i * _UB
    c = c_ref[...]
    x = x_ref[...]
    lane = lax.broadcasted_iota(jnp.int32, (1, _UB), 1)
    c = jnp.where(lane < valid, c, 0.0)
    x = jnp.where(lane.reshape(_UB, 1) < valid, x, 0.0)
    acc_ref[...] += lax.dot_general(
        c, x, (((1,), (0,)), ((), ())),
        preferred_element_type=jnp.float32, precision=_HI)

    @pl.when(i == pl.num_programs(0) - 1)
    def _():
        c8 = acc_ref[...]
        cs = c8[:_AR] + c8[_AR:]
        o_ref[...] = jnp.dot(cs, w1_ref[...], preferred_element_type=jnp.float32,
                             precision=_HI)


def _out_call(c2, node_embs, W1):
    return pl.pallas_call(
        _out_body,
        grid=(_UG,),
        in_specs=[
            pl.BlockSpec((_NC * _AR, _UB), lambda i: (0, i)),
            pl.BlockSpec((_UB, _F), lambda i: (i, 0)),
            pl.BlockSpec((_F, _F), lambda i: (0, 0)),
        ],
        out_specs=pl.BlockSpec((_AR, _F), lambda i: (0, 0)),
        out_shape=jax.ShapeDtypeStruct((_AR, _F), jnp.float32),
        scratch_shapes=[pltpu.VMEM((_NC * _AR, _F), jnp.float32)],
    )(c2, node_embs, W1)


def kernel(node_embs, edge_embs, edge_list, W1, W2, a1, a2):
    del edge_embs, W2, a2
    a1r = jnp.broadcast_to(a1.reshape(1, _F), (8, _F))
    u8 = _u_call(node_embs, W1, a1r)
    u = u8.reshape(_UG, 8, _UB)[:, 0, :].reshape(_NP)
    el = edge_list.astype(jnp.int32).T
    el = jnp.pad(el, ((0, 0), (0, _MPAD - _M)))
    el3 = el.reshape(_AR, _NW, _EPW)
    si = el + (jnp.arange(_AR, dtype=jnp.int32) * _N)[:, None]
    si4 = si.reshape(_AR, _NW, _NCH, _CHUNK)
    zeros_c = jnp.zeros((_BUF,), jnp.float32)
    c2 = _sc_call(u, el3, si4, zeros_c)
    return _out_call(c2.reshape(_NC * _AR, _N), node_embs, W1)


# speedup vs baseline: 7.6319x; 1.0019x over previous
import functools

import jax
import jax.numpy as jnp
from jax import lax
from jax.experimental import pallas as pl
from jax.experimental.pallas import tpu as pltpu
from jax.experimental.pallas import tpu_sc as plsc

_N = 100000
_M = 100000
_AR = 4
_F = 128
_ALPHA = 0.2

_NC = 2
_NS = 16
_NW = _NC * _NS
_CHUNK = 128
_NCH = 25
_EPW = _NCH * _CHUNK
_MPAD = _NW * _EPW
_CSLICE = _AR * _N // _NS
_BUF = _AR * _EPW

_HI = lax.Precision.HIGHEST
_HP = lax.Precision.HIGH
_UB = 2048
_UG = (_N + _UB - 1) // _UB
_NP = _UG * _UB


def _split(x):
    hi = x.astype(jnp.bfloat16)
    lo = (x - hi.astype(jnp.float32)).astype(jnp.bfloat16)
    return hi, lo


def _u_body(x_ref, w1_ref, a1r_ref, u_ref):
    xh, xl = _split(x_ref[...])
    wh_, wl_ = _split(w1_ref[...])
    d = functools.partial(jnp.dot, preferred_element_type=jnp.float32)
    wh = d(xh, wh_) + d(xh, wl_) + d(xl, wh_)
    lr = jnp.where(wh >= 0.0, wh, _ALPHA * wh)
    lh, ll = _split(lr)
    ah, al = _split(a1r_ref[...])
    dg = functools.partial(lax.dot_general,
                           dimension_numbers=(((1,), (1,)), ((), ())),
                           preferred_element_type=jnp.float32)
    u_ref[...] = dg(ah, lh) + dg(al, lh) + dg(ah, ll)


def _u_call(node_embs, W1, a1r):
    return pl.pallas_call(
        _u_body,
        grid=(_UG,),
        in_specs=[
            pl.BlockSpec((_UB, _F), lambda i: (i, 0)),
            pl.BlockSpec((_F, _F), lambda i: (0, 0)),
            pl.BlockSpec((8, _F), lambda i: (0, 0)),
        ],
        out_specs=pl.BlockSpec((8, _UB), lambda i: (i, 0)),
        out_shape=jax.ShapeDtypeStruct((_UG * 8, _UB), jnp.float32),
    )(node_embs, W1, a1r)


def _sc_body(u_hbm, el_hbm, si_hbm, zeros_hbm, c2_hbm,
             ei_v, si_v, w_v, gu_v, c_sh, u_sh, sem):
    cid = lax.axis_index("c")
    sid = lax.axis_index("s")
    wid = cid * _NS + sid

    pltpu.sync_copy(zeros_hbm, w_v)
    base_c = sid * _CSLICE
    pltpu.sync_copy(w_v, c_sh.at[pl.ds(base_c, _BUF)])
    pltpu.sync_copy(w_v.at[pl.ds(0, _CSLICE - _BUF)],
                    c_sh.at[pl.ds(base_c + _BUF, _CSLICE - _BUF)])

    upt = _NP // _NS
    pltpu.sync_copy(u_hbm.at[pl.ds(sid * upt, upt)], gu_v.at[pl.ds(0, upt)])
    pltpu.sync_copy(gu_v.at[pl.ds(0, upt)], u_sh.at[pl.ds(sid * upt, upt)])

    for k in range(_AR):
        pltpu.sync_copy(el_hbm.at[k, wid], ei_v.at[pl.ds(k * _EPW, _EPW)])
        pltpu.sync_copy(si_hbm.at[k, wid], si_v.at[pl.ds(k * _NCH, _NCH)])

    plsc.subcore_barrier()

    def g_body(r, carry):
        sl = pl.ds(r * _CHUNK, _CHUNK)
        pltpu.sync_copy(u_sh.at[ei_v.at[sl]], gu_v.at[sl])
        return carry

    lax.fori_loop(0, _AR * _NCH, g_body, 0)

    base = wid * _EPW
    iota16 = lax.iota(jnp.int32, 16)

    def chunk_body(c, carry):
        pos = c * 16 + iota16
        g = [plsc.load_gather(gu_v, [pos + k * _EPW]) for k in range(_AR)]
        mx = jnp.maximum(jnp.maximum(g[0], g[1]), jnp.maximum(g[2], g[3]))
        e = [jnp.exp(gk - mx) for gk in g]
        inv = 1.0 / (e[0] + e[1] + e[2] + e[3])
        valid = (base + c * 16 + iota16) < _M
        for k in range(_AR):
            plsc.store_scatter(w_v, [pos + k * _EPW],
                               jnp.where(valid, e[k] * inv, 0.0))
        return carry

    lax.fori_loop(0, _EPW // 16, chunk_body, 0)

    def scat_body(r, carry):
        pltpu.sync_copy(w_v.at[pl.ds(r * _CHUNK, _CHUNK)],
                        c_sh.at[si_v.at[r]], add=True)
        return carry

    lax.fori_loop(0, _AR * _NCH, scat_body, 0)

    plsc.subcore_barrier()

    hbase = cid * _AR * _N + sid * _CSLICE
    pltpu.sync_copy(c_sh.at[pl.ds(base_c, _BUF)], w_v)
    pltpu.sync_copy(w_v, c2_hbm.at[pl.ds(hbase, _BUF)])
    rem = _CSLICE - _BUF
    pltpu.sync_copy(c_sh.at[pl.ds(base_c + _BUF, rem)], w_v.at[pl.ds(0, rem)])
    pltpu.sync_copy(w_v.at[pl.ds(0, rem)], c2_hbm.at[pl.ds(hbase + _BUF, rem)])


def _sc_call(u_flat, el3, si4, zeros_c):
    mesh = plsc.VectorSubcoreMesh(core_axis_name="c", subcore_axis_name="s",
                                  num_cores=_NC, num_subcores=_NS)
    f = functools.partial(
        pl.kernel,
        out_type=jax.ShapeDtypeStruct((_NC * _AR * _N,), jnp.float32),
        mesh=mesh,
        compiler_params=pltpu.CompilerParams(needs_layout_passes=False),
        scratch_types=[
            pltpu.VMEM((_AR * _EPW,), jnp.int32),
            pltpu.VMEM((_AR * _NCH, _CHUNK), jnp.int32),
            pltpu.VMEM((_AR * _EPW,), jnp.float32),
            pltpu.VMEM((_AR * _EPW,), jnp.float32),
            pltpu.VMEM_SHARED((_AR * _N,), jnp.float32),
            pltpu.VMEM_SHARED((_NP,), jnp.float32),
            pltpu.SemaphoreType.DMA,
        ],
    )(_sc_body)
    return f(u_flat, el3, si4, zeros_c)


def _out_body(c_ref, x_ref, w1_ref, o_ref, acc_ref):
    i = pl.program_id(0)

    @pl.when(i == 0)
    def _():
        acc_ref[...] = jnp.zeros_like(acc_ref)

    valid = _N - i * _UB
    c = c_ref[...]
    x = x_ref[...]
    lane = lax.broadcasted_iota(jnp.int32, (1, _UB), 1)
    c = jnp.where(lane < valid, c, 0.0)
    x = jnp.where(lane.reshape(_UB, 1) < valid, x, 0.0)
    ch, cl = _split(c)
    xh, xl = _split(x)
    dg = functools.partial(lax.dot_general,
                           dimension_numbers=(((1,), (0,)), ((), ())),
                           preferred_element_type=jnp.float32)
    acc_ref[...] += dg(ch, xh) + dg(ch, xl) + dg(cl, xh)

    @pl.when(i == pl.num_programs(0) - 1)
    def _():
        c8 = acc_ref[...]
        cs = c8[:_AR] + c8[_AR:]
        o_ref[...] = jnp.dot(cs, w1_ref[...], preferred_element_type=jnp.float32,
                             precision=_HI)


def _out_call(c2, node_embs, W1):
    return pl.pallas_call(
        _out_body,
        grid=(_UG,),
        in_specs=[
            pl.BlockSpec((_NC * _AR, _UB), lambda i: (0, i)),
            pl.BlockSpec((_UB, _F), lambda i: (i, 0)),
            pl.BlockSpec((_F, _F), lambda i: (0, 0)),
        ],
        out_specs=pl.BlockSpec((_AR, _F), lambda i: (0, 0)),
        out_shape=jax.ShapeDtypeStruct((_AR, _F), jnp.float32),
        scratch_shapes=[pltpu.VMEM((_NC * _AR, _F), jnp.float32)],
    )(c2, node_embs, W1)


def kernel(node_embs, edge_embs, edge_list, W1, W2, a1, a2):
    del edge_embs, W2, a2
    a1r = jnp.broadcast_to(a1.reshape(1, _F), (8, _F))
    u8 = _u_call(node_embs, W1, a1r)
    u = u8.reshape(_UG, 8, _UB)[:, 0, :].reshape(_NP)
    el = edge_list.astype(jnp.int32).T
    el = jnp.pad(el, ((0, 0), (0, _MPAD - _M)))
    el3 = el.reshape(_AR, _NW, _EPW)
    si = el + (jnp.arange(_AR, dtype=jnp.int32) * _N)[:, None]
    si4 = si.reshape(_AR, _NW, _NCH, _CHUNK)
    zeros_c = jnp.zeros((_BUF,), jnp.float32)
    c2 = _sc_call(u, el3, si4, zeros_c)
    return _out_call(c2.reshape(_NC * _AR, _N), node_embs, W1)
